# baseline (speedup 1.0000x reference)
import functools

import jax
import jax.numpy as jnp
from jax import lax
from jax.experimental import pallas as pl
from jax.experimental.pallas import tpu as pltpu
from jax.experimental.pallas import tpu_sc as plsc

NC = 2
NS = 16
NW = NC * NS
CK = 128
NB = 2
L = 16

_SC_PARAMS = pltpu.CompilerParams(needs_layout_passes=False)


def _mesh():
  return plsc.VectorSubcoreMesh(core_axis_name="c", subcore_axis_name="s")


SB = 40
C0_FRAC = 0.75


def _split_chunks(tot):
  c0 = 8 * int(round(C0_FRAC * tot / 8))
  c0 = min(max(c0, 8), tot - 8)
  return c0, tot - c0


def _blocks(c):
  out = []
  while c > 0:
    s = min(c, SB)
    out.append(s)
    c -= s
  return out


def _sc_aggregate(h, src3, dst3, zeros_hbm, n_pad, tot):
  d = h.shape[1]
  rows_per_tile = n_pad // NS
  c0, c1 = _split_chunks(tot)

  @functools.partial(
      pl.kernel,
      mesh=_mesh(),
      compiler_params=_SC_PARAMS,
      out_type=jax.ShapeDtypeStruct((NC, n_pad, d), jnp.float32),
      scratch_types=[
          pltpu.VMEM((SB, CK), jnp.int32),
          pltpu.VMEM((SB, CK), jnp.int32),
          pltpu.VMEM((NB, CK, d), jnp.float32),
          pltpu.VMEM_SHARED((n_pad, d), jnp.float32),
          [pltpu.SemaphoreType.DMA] * NB,
          [pltpu.SemaphoreType.DMA] * NB,
      ],
  )
  def k(h_hbm, src_hbm, dst_hbm, z_hbm, agg_out,
        src_v, dst_v, rows_v, agg_sp, gsems, ssems):
    cid = lax.axis_index("c")
    sid = lax.axis_index("s")

    pltpu.sync_copy(z_hbm, agg_sp.at[pl.ds(sid * rows_per_tile, rows_per_tile)])
    plsc.subcore_barrier()

    def gather(j, b):
      return pltpu.make_async_copy(h_hbm.at[src_v.at[j]], rows_v.at[b],
                                   gsems[b])

    def scatter(j, b):
      return pltpu.make_async_copy(rows_v.at[b], agg_sp.at[dst_v.at[j]],
                                   ssems[b])

    def run_block(base, sbc):
      pltpu.sync_copy(src_hbm.at[sid, pl.ds(base, sbc)],
                      src_v.at[pl.ds(0, sbc)])
      pltpu.sync_copy(dst_hbm.at[sid, pl.ds(base, sbc)],
                      dst_v.at[pl.ds(0, sbc)])

      gather(0, 0).start()
      gather(0, 0).wait()
      scatter(0, 0).start(add=True)
      gather(1, 1).start()

      def body(g, carry):
        for b, j in ((1, 2 * g + 1), (0, 2 * g + 2)):
          gather(j, b).wait()
          scatter(j, b).start(add=True)
          nb = 1 - b
          scatter(j - 1, nb).wait()
          gather(j + 1, nb).start()
        return carry
      lax.fori_loop(0, sbc // 2 - 1, body, 0)

      jl = sbc - 1
      gather(jl, 1).wait()
      scatter(jl, 1).start(add=True)
      scatter(jl - 1, 0).wait()
      scatter(jl, 1).wait()

    @pl.when(cid == 0)
    def _():
      base = 0
      for sbc in _blocks(c0):
        run_block(base, sbc)
        base += sbc

    @pl.when(cid == 1)
    def _():
      base = c0
      for sbc in _blocks(c1):
        run_block(base, sbc)
        base += sbc

    plsc.subcore_barrier()
    pltpu.sync_copy(agg_sp.at[pl.ds(sid * rows_per_tile, rows_per_tile)],
                    agg_out.at[cid, pl.ds(sid * rows_per_tile, rows_per_tile)])

  return k(h, src3, dst3, zeros_hbm)


def _sc_degrees(dstf0, dstf1, n_pad, ew):
  @functools.partial(
      pl.kernel,
      mesh=_mesh(),
      compiler_params=_SC_PARAMS,
      out_type=[
          jax.ShapeDtypeStruct((NW, 1, n_pad), jnp.float32),
          jax.ShapeDtypeStruct((NW, 1, n_pad), jnp.float32),
      ],
      scratch_types=[
          pltpu.VMEM((ew,), jnp.int32),
          pltpu.VMEM((n_pad,), jnp.float32),
      ],
  )
  def k(d0_hbm, d1_hbm, deg0_out, deg1_out, dst_v, deg_v):
    cid = lax.axis_index("c")
    sid = lax.axis_index("s")
    wid = cid * NS + sid

    zero16 = jnp.zeros((L,), jnp.float32)
    one16 = jnp.ones((L,), jnp.float32)

    for d_hbm, out in ((d0_hbm, deg0_out), (d1_hbm, deg1_out)):
      pltpu.sync_copy(d_hbm.at[pl.ds(wid * ew, ew)], dst_v)

      def zdeg(i, carry):
        deg_v[pl.ds(i * L, L)] = zero16
        return carry
      lax.fori_loop(0, n_pad // L, zdeg, 0)

      def accum(i, carry):
        idx = dst_v[pl.ds(i * L, L)]
        plsc.addupdate_scatter(deg_v, [idx], one16)
        return carry
      lax.fori_loop(0, ew // L, accum, 0)

      pltpu.sync_copy(deg_v, out.at[wid, 0])

  return k(dstf0, dstf1)


def _tc_layer(h, agg, deg_t, w_self, w_neigh, b, relu):
  n, d = h.shape
  bn = 400
  grid = (n // bn,)

  def body(h_ref, agg_ref, deg_ref, ws_ref, wn_ref, b_ref, o_ref):
    a = agg_ref[0] + agg_ref[1]
    deg = jnp.sum(deg_ref[...], axis=1, keepdims=True)
    recip = 1.0 / jnp.clip(deg, 1.0, None)
    out = (
        jnp.dot(h_ref[...], ws_ref[...], preferred_element_type=jnp.float32)
        + jnp.dot(a, wn_ref[...], preferred_element_type=jnp.float32) * recip
        + b_ref[...]
    )
    if relu:
      out = jnp.maximum(out, 0.0)
    o_ref[...] = out

  return pl.pallas_call(
      body,
      grid=grid,
      in_specs=[
          pl.BlockSpec((bn, d), lambda i: (i, 0)),
          pl.BlockSpec((NC, bn, d), lambda i: (0, i, 0)),
          pl.BlockSpec((bn, NW), lambda i: (i, 0)),
          pl.BlockSpec((d, d), lambda i: (0, 0)),
          pl.BlockSpec((d, d), lambda i: (0, 0)),
          pl.BlockSpec((1, d), lambda i: (0, 0)),
      ],
      out_specs=pl.BlockSpec((bn, d), lambda i: (i, 0)),
      out_shape=jax.ShapeDtypeStruct((n, d), jnp.float32),
  )(h, agg, deg_t, w_self, w_neigh, b)


def _prep_edges(edge_index, n_nodes):
  e = edge_index.shape[1]
  tot = -(-e // (NS * CK * SB)) * SB
  e_pad = NS * tot * CK
  src = jnp.concatenate(
      [edge_index[0], jnp.zeros((e_pad - e,), jnp.int32)])
  dst = jnp.concatenate(
      [edge_index[1], jnp.full((e_pad - e,), n_nodes, jnp.int32)])
  dst_flat = jnp.concatenate([dst, jnp.zeros((8,), jnp.int32)])
  return (src.reshape(NS, tot, CK), dst.reshape(NS, tot, CK),
          dst_flat, tot)


def kernel(x, edge_index0, edge_index1, W_self0, W_neigh0, b0,
           W_self1, W_neigh1, b1):
  n, d = x.shape
  n_pad = -(-(n + 1) // 128) * 128
  zeros_hbm = jnp.zeros((n_pad // NS, d), jnp.float32)

  src0, dst0, dstf0, tot = _prep_edges(edge_index0, n)
  src1, dst1, dstf1, _ = _prep_edges(edge_index1, n)
  ew = tot * CK * NS // NW

  deg0, deg1 = _sc_degrees(dstf0, dstf1, n_pad, ew)
  deg0_t = deg0.reshape(NW, n_pad).T[:n]
  deg1_t = deg1.reshape(NW, n_pad).T[:n]

  agg0 = _sc_aggregate(x, src0, dst0, zeros_hbm, n_pad, tot)
  h = _tc_layer(x, agg0, deg0_t, W_self0, W_neigh0, b0.reshape(1, -1),
                relu=True)
  agg1 = _sc_aggregate(h, src1, dst1, zeros_hbm, n_pad, tot)
  return _tc_layer(h, agg1, deg1_t, W_self1, W_neigh1, b1.reshape(1, -1),
                   relu=False)


# speedup vs baseline: 1.0103x; 1.0103x over previous
import functools

import jax
import jax.numpy as jnp
from jax import lax
from jax.experimental import pallas as pl
from jax.experimental.pallas import tpu as pltpu
from jax.experimental.pallas import tpu_sc as plsc

NC = 2
NS = 16
NW = NC * NS
CK = 64
NB = 4
L = 16

_SC_PARAMS = pltpu.CompilerParams(needs_layout_passes=False)


def _mesh():
  return plsc.VectorSubcoreMesh(core_axis_name="c", subcore_axis_name="s")


SB = 64
C0_FRAC = 0.75


def _split_chunks(tot):
  c0 = 16 * int(round(C0_FRAC * tot / 16))
  c0 = min(max(c0, 16), tot - 16)
  return c0, tot - c0


def _blocks(c):
  out = []
  while c > 0:
    s = min(c, SB)
    out.append(s)
    c -= s
  return out


def _sc_aggregate(h, src3, dst3, zeros_hbm, n_pad, tot):
  d = h.shape[1]
  rows_per_tile = n_pad // NS
  c0, c1 = _split_chunks(tot)

  @functools.partial(
      pl.kernel,
      mesh=_mesh(),
      compiler_params=_SC_PARAMS,
      out_type=jax.ShapeDtypeStruct((NC, n_pad, d), jnp.float32),
      scratch_types=[
          pltpu.VMEM((SB, CK), jnp.int32),
          pltpu.VMEM((SB, CK), jnp.int32),
          pltpu.VMEM((NB, CK, d), jnp.float32),
          pltpu.VMEM_SHARED((n_pad, d), jnp.float32),
          [pltpu.SemaphoreType.DMA] * NB,
          [pltpu.SemaphoreType.DMA] * NB,
      ],
  )
  def k(h_hbm, src_hbm, dst_hbm, z_hbm, agg_out,
        src_v, dst_v, rows_v, agg_sp, gsems, ssems):
    cid = lax.axis_index("c")
    sid = lax.axis_index("s")

    pltpu.sync_copy(z_hbm, agg_sp.at[pl.ds(sid * rows_per_tile, rows_per_tile)])
    plsc.subcore_barrier()

    def gather(j, b):
      return pltpu.make_async_copy(h_hbm.at[src_v.at[j]], rows_v.at[b],
                                   gsems[b])

    def scatter(j, b):
      return pltpu.make_async_copy(rows_v.at[b], agg_sp.at[dst_v.at[j]],
                                   ssems[b])

    def run_block(base, sbc):
      pltpu.sync_copy(src_hbm.at[sid, pl.ds(base, sbc)],
                      src_v.at[pl.ds(0, sbc)])
      pltpu.sync_copy(dst_hbm.at[sid, pl.ds(base, sbc)],
                      dst_v.at[pl.ds(0, sbc)])

      gather(0, 0).start()
      gather(1, 1).start()
      for j in (0, 1):
        gather(j, j).wait()
        scatter(j, j).start(add=True)
        gather(j + 2, (j + 2) % NB).start()

      def body(q, carry):
        for u in range(4):
          j = 4 * q + 2 + u
          b = (2 + u) % NB
          gather(j, b).wait()
          scatter(j, b).start(add=True)
          scatter(j - 2, (b + 2) % NB).wait()
          gather(j + 2, (b + 2) % NB).start()
        return carry
      lax.fori_loop(0, (sbc - 4) // 4, body, 0)

      for j in (sbc - 2, sbc - 1):
        b = j % NB
        gather(j, b).wait()
        scatter(j, b).start(add=True)
        scatter(j - 2, (b + 2) % NB).wait()
      scatter(sbc - 2, (sbc - 2) % NB).wait()
      scatter(sbc - 1, (sbc - 1) % NB).wait()

    @pl.when(cid == 0)
    def _():
      base = 0
      for sbc in _blocks(c0):
        run_block(base, sbc)
        base += sbc

    @pl.when(cid == 1)
    def _():
      base = c0
      for sbc in _blocks(c1):
        run_block(base, sbc)
        base += sbc

    plsc.subcore_barrier()
    pltpu.sync_copy(agg_sp.at[pl.ds(sid * rows_per_tile, rows_per_tile)],
                    agg_out.at[cid, pl.ds(sid * rows_per_tile, rows_per_tile)])

  return k(h, src3, dst3, zeros_hbm)


def _sc_degrees(dstf0, dstf1, n_pad, ew):
  @functools.partial(
      pl.kernel,
      mesh=_mesh(),
      compiler_params=_SC_PARAMS,
      out_type=[
          jax.ShapeDtypeStruct((NW, 1, n_pad), jnp.float32),
          jax.ShapeDtypeStruct((NW, 1, n_pad), jnp.float32),
      ],
      scratch_types=[
          pltpu.VMEM((ew,), jnp.int32),
          pltpu.VMEM((n_pad,), jnp.float32),
      ],
  )
  def k(d0_hbm, d1_hbm, deg0_out, deg1_out, dst_v, deg_v):
    cid = lax.axis_index("c")
    sid = lax.axis_index("s")
    wid = cid * NS + sid

    zero16 = jnp.zeros((L,), jnp.float32)
    one16 = jnp.ones((L,), jnp.float32)

    for d_hbm, out in ((d0_hbm, deg0_out), (d1_hbm, deg1_out)):
      pltpu.sync_copy(d_hbm.at[pl.ds(wid * ew, ew)], dst_v)

      def zdeg(i, carry):
        deg_v[pl.ds(i * L, L)] = zero16
        return carry
      lax.fori_loop(0, n_pad // L, zdeg, 0)

      def accum(i, carry):
        idx = dst_v[pl.ds(i * L, L)]
        plsc.addupdate_scatter(deg_v, [idx], one16)
        return carry
      lax.fori_loop(0, ew // L, accum, 0)

      pltpu.sync_copy(deg_v, out.at[wid, 0])

  return k(dstf0, dstf1)


def _tc_layer(h, agg, deg_t, w_self, w_neigh, b, relu):
  n, d = h.shape
  bn = 400
  grid = (n // bn,)

  def body(h_ref, agg_ref, deg_ref, ws_ref, wn_ref, b_ref, o_ref):
    a = agg_ref[0] + agg_ref[1]
    deg = jnp.sum(deg_ref[...], axis=1, keepdims=True)
    recip = 1.0 / jnp.clip(deg, 1.0, None)
    out = (
        jnp.dot(h_ref[...], ws_ref[...], preferred_element_type=jnp.float32)
        + jnp.dot(a, wn_ref[...], preferred_element_type=jnp.float32) * recip
        + b_ref[...]
    )
    if relu:
      out = jnp.maximum(out, 0.0)
    o_ref[...] = out

  return pl.pallas_call(
      body,
      grid=grid,
      in_specs=[
          pl.BlockSpec((bn, d), lambda i: (i, 0)),
          pl.BlockSpec((NC, bn, d), lambda i: (0, i, 0)),
          pl.BlockSpec((bn, NW), lambda i: (i, 0)),
          pl.BlockSpec((d, d), lambda i: (0, 0)),
          pl.BlockSpec((d, d), lambda i: (0, 0)),
          pl.BlockSpec((1, d), lambda i: (0, 0)),
      ],
      out_specs=pl.BlockSpec((bn, d), lambda i: (i, 0)),
      out_shape=jax.ShapeDtypeStruct((n, d), jnp.float32),
  )(h, agg, deg_t, w_self, w_neigh, b)


def _prep_edges(edge_index, n_nodes):
  e = edge_index.shape[1]
  tot = -(-e // (NS * CK * SB)) * SB
  e_pad = NS * tot * CK
  src = jnp.concatenate(
      [edge_index[0], jnp.zeros((e_pad - e,), jnp.int32)])
  dst = jnp.concatenate(
      [edge_index[1], jnp.full((e_pad - e,), n_nodes, jnp.int32)])
  dst_flat = jnp.concatenate([dst, jnp.zeros((8,), jnp.int32)])
  return (src.reshape(NS, tot, CK), dst.reshape(NS, tot, CK),
          dst_flat, tot)


def kernel(x, edge_index0, edge_index1, W_self0, W_neigh0, b0,
           W_self1, W_neigh1, b1):
  n, d = x.shape
  n_pad = -(-(n + 1) // 128) * 128
  zeros_hbm = jnp.zeros((n_pad // NS, d), jnp.float32)

  src0, dst0, dstf0, tot = _prep_edges(edge_index0, n)
  src1, dst1, dstf1, _ = _prep_edges(edge_index1, n)
  ew = tot * CK * NS // NW

  deg0, deg1 = _sc_degrees(dstf0, dstf1, n_pad, ew)
  deg0_t = deg0.reshape(NW, n_pad).T[:n]
  deg1_t = deg1.reshape(NW, n_pad).T[:n]

  agg0 = _sc_aggregate(x, src0, dst0, zeros_hbm, n_pad, tot)
  h = _tc_layer(x, agg0, deg0_t, W_self0, W_neigh0, b0.reshape(1, -1),
                relu=True)
  agg1 = _sc_aggregate(h, src1, dst1, zeros_hbm, n_pad, tot)
  return _tc_layer(h, agg1, deg1_t, W_self1, W_neigh1, b1.reshape(1, -1),
                   relu=False)


# speedup vs baseline: 1.0170x; 1.0067x over previous
import functools

import jax
import jax.numpy as jnp
from jax import lax
from jax.experimental import pallas as pl
from jax.experimental.pallas import tpu as pltpu
from jax.experimental.pallas import tpu_sc as plsc

NC = 2
NS = 16
NW = NC * NS
CK = 64
NB = 4
L = 16

_SC_PARAMS = pltpu.CompilerParams(needs_layout_passes=False)


def _mesh():
  return plsc.VectorSubcoreMesh(core_axis_name="c", subcore_axis_name="s")


SB = 64
C0_FRAC = 0.75


def _split_chunks(tot):
  c0 = 16 * int(round(C0_FRAC * tot / 16))
  c0 = min(max(c0, 16), tot - 16)
  return c0, tot - c0


def _blocks(c):
  out = []
  while c > 0:
    s = min(c, SB)
    out.append(s)
    c -= s
  return out


def _sc_aggregate(h, src3, dst3, zeros_hbm, n_pad, tot):
  d = h.shape[1]
  rows_per_tile = n_pad // NS
  c0, c1 = _split_chunks(tot)

  @functools.partial(
      pl.kernel,
      mesh=_mesh(),
      compiler_params=_SC_PARAMS,
      out_type=jax.ShapeDtypeStruct((NC, n_pad, d), jnp.float32),
      scratch_types=[
          pltpu.VMEM((SB, CK), jnp.int32),
          pltpu.VMEM((SB, CK), jnp.int32),
          pltpu.VMEM((NB, CK, d), jnp.float32),
          pltpu.VMEM_SHARED((n_pad, d), jnp.float32),
          [pltpu.SemaphoreType.DMA] * NB,
          [pltpu.SemaphoreType.DMA] * NB,
      ],
  )
  def k(h_hbm, src_hbm, dst_hbm, z_hbm, agg_out,
        src_v, dst_v, rows_v, agg_sp, gsems, ssems):
    cid = lax.axis_index("c")
    sid = lax.axis_index("s")
    row0 = sid * rows_per_tile

    pltpu.sync_copy(z_hbm, rows_v.at[0])
    for i in range(rows_per_tile // CK):
      pltpu.sync_copy(rows_v.at[0], agg_sp.at[pl.ds(row0 + i * CK, CK)])
    rem = rows_per_tile % CK
    if rem:
      pltpu.sync_copy(
          rows_v.at[0, pl.ds(0, rem)],
          agg_sp.at[pl.ds(row0 + (rows_per_tile // CK) * CK, rem)])
    plsc.subcore_barrier()

    def gather(j, b):
      return pltpu.make_async_copy(h_hbm.at[src_v.at[j]], rows_v.at[b],
                                   gsems[b])

    def scatter(j, b):
      return pltpu.make_async_copy(rows_v.at[b], agg_sp.at[dst_v.at[j]],
                                   ssems[b])

    def run_block(base, sbc):
      pltpu.sync_copy(src_hbm.at[sid, pl.ds(base, sbc)],
                      src_v.at[pl.ds(0, sbc)])
      pltpu.sync_copy(dst_hbm.at[sid, pl.ds(base, sbc)],
                      dst_v.at[pl.ds(0, sbc)])

      gather(0, 0).start()
      gather(1, 1).start()
      for j in (0, 1):
        gather(j, j).wait()
        scatter(j, j).start(add=True)
        gather(j + 2, (j + 2) % NB).start()

      def body(q, carry):
        for u in range(4):
          j = 4 * q + 2 + u
          b = (2 + u) % NB
          gather(j, b).wait()
          scatter(j, b).start(add=True)
          scatter(j - 2, (b + 2) % NB).wait()
          gather(j + 2, (b + 2) % NB).start()
        return carry
      lax.fori_loop(0, (sbc - 4) // 4, body, 0)

      for j in (sbc - 2, sbc - 1):
        b = j % NB
        gather(j, b).wait()
        scatter(j, b).start(add=True)
        scatter(j - 2, (b + 2) % NB).wait()
      scatter(sbc - 2, (sbc - 2) % NB).wait()
      scatter(sbc - 1, (sbc - 1) % NB).wait()

    @pl.when(cid == 0)
    def _():
      base = 0
      for sbc in _blocks(c0):
        run_block(base, sbc)
        base += sbc

    @pl.when(cid == 1)
    def _():
      base = c0
      for sbc in _blocks(c1):
        run_block(base, sbc)
        base += sbc

    plsc.subcore_barrier()
    pltpu.sync_copy(agg_sp.at[pl.ds(sid * rows_per_tile, rows_per_tile)],
                    agg_out.at[cid, pl.ds(sid * rows_per_tile, rows_per_tile)])

  return k(h, src3, dst3, zeros_hbm)


def _sc_degrees(dstf0, dstf1, n_pad, ew):
  @functools.partial(
      pl.kernel,
      mesh=_mesh(),
      compiler_params=_SC_PARAMS,
      out_type=[
          jax.ShapeDtypeStruct((NW, 1, n_pad), jnp.float32),
          jax.ShapeDtypeStruct((NW, 1, n_pad), jnp.float32),
      ],
      scratch_types=[
          pltpu.VMEM((ew,), jnp.int32),
          pltpu.VMEM((n_pad,), jnp.float32),
      ],
  )
  def k(d0_hbm, d1_hbm, deg0_out, deg1_out, dst_v, deg_v):
    cid = lax.axis_index("c")
    sid = lax.axis_index("s")
    wid = cid * NS + sid

    zero16 = jnp.zeros((L,), jnp.float32)
    one16 = jnp.ones((L,), jnp.float32)

    for d_hbm, out in ((d0_hbm, deg0_out), (d1_hbm, deg1_out)):
      pltpu.sync_copy(d_hbm.at[pl.ds(wid * ew, ew)], dst_v)

      def zdeg(i, carry):
        deg_v[pl.ds(i * L, L)] = zero16
        return carry
      lax.fori_loop(0, n_pad // L, zdeg, 0)

      def accum(i, carry):
        idx = dst_v[pl.ds(i * L, L)]
        plsc.addupdate_scatter(deg_v, [idx], one16)
        return carry
      lax.fori_loop(0, ew // L, accum, 0)

      pltpu.sync_copy(deg_v, out.at[wid, 0])

  return k(dstf0, dstf1)


def _tc_layer(h, agg, deg_t, w_self, w_neigh, b, relu):
  n, d = h.shape
  bn = 400
  grid = (n // bn,)

  def body(h_ref, agg_ref, deg_ref, ws_ref, wn_ref, b_ref, o_ref):
    a = agg_ref[0] + agg_ref[1]
    deg = jnp.sum(deg_ref[...], axis=1, keepdims=True)
    recip = 1.0 / jnp.clip(deg, 1.0, None)
    out = (
        jnp.dot(h_ref[...], ws_ref[...], preferred_element_type=jnp.float32)
        + jnp.dot(a, wn_ref[...], preferred_element_type=jnp.float32) * recip
        + b_ref[...]
    )
    if relu:
      out = jnp.maximum(out, 0.0)
    o_ref[...] = out

  return pl.pallas_call(
      body,
      grid=grid,
      in_specs=[
          pl.BlockSpec((bn, d), lambda i: (i, 0)),
          pl.BlockSpec((NC, bn, d), lambda i: (0, i, 0)),
          pl.BlockSpec((bn, NW), lambda i: (i, 0)),
          pl.BlockSpec((d, d), lambda i: (0, 0)),
          pl.BlockSpec((d, d), lambda i: (0, 0)),
          pl.BlockSpec((1, d), lambda i: (0, 0)),
      ],
      out_specs=pl.BlockSpec((bn, d), lambda i: (i, 0)),
      out_shape=jax.ShapeDtypeStruct((n, d), jnp.float32),
  )(h, agg, deg_t, w_self, w_neigh, b)


def _prep_edges(edge_index, n_nodes):
  e = edge_index.shape[1]
  tot = -(-e // (NS * CK * SB)) * SB
  e_pad = NS * tot * CK
  src = jnp.concatenate(
      [edge_index[0], jnp.zeros((e_pad - e,), jnp.int32)])
  dst = jnp.concatenate(
      [edge_index[1], jnp.full((e_pad - e,), n_nodes, jnp.int32)])
  dst_flat = jnp.concatenate([dst, jnp.zeros((8,), jnp.int32)])
  return (src.reshape(NS, tot, CK), dst.reshape(NS, tot, CK),
          dst_flat, tot)


def kernel(x, edge_index0, edge_index1, W_self0, W_neigh0, b0,
           W_self1, W_neigh1, b1):
  n, d = x.shape
  n_pad = -(-(n + 1) // 128) * 128
  zeros_hbm = jnp.zeros((CK, d), jnp.float32)

  src0, dst0, dstf0, tot = _prep_edges(edge_index0, n)
  src1, dst1, dstf1, _ = _prep_edges(edge_index1, n)
  ew = tot * CK * NS // NW

  deg0, deg1 = _sc_degrees(dstf0, dstf1, n_pad, ew)
  deg0_t = deg0.reshape(NW, n_pad).T[:n]
  deg1_t = deg1.reshape(NW, n_pad).T[:n]

  agg0 = _sc_aggregate(x, src0, dst0, zeros_hbm, n_pad, tot)
  h = _tc_layer(x, agg0, deg0_t, W_self0, W_neigh0, b0.reshape(1, -1),
                relu=True)
  agg1 = _sc_aggregate(h, src1, dst1, zeros_hbm, n_pad, tot)
  return _tc_layer(h, agg1, deg1_t, W_self1, W_neigh1, b1.reshape(1, -1),
                   relu=False)
